# baseline (device time: 135547 ns/iter reference)
import jax
import jax.numpy as jnp
from jax import lax
from jax.experimental import pallas as pl
from jax.experimental.pallas import tpu as pltpu

N_DEV = 4
T = 1024
D = 1024
F = 2048
E = 16
E_LOCAL = E // N_DEV
T_SHARD = T // N_DEV
FBLK = 512


def _ag_body(x_ref, r_ref, xout_ref, rout_ref,
             x_send_sems, x_recv_sems, r_send_sems, r_recv_sems):
    my = lax.axis_index("i")
    left = lax.rem(my + N_DEV - 1, N_DEV)
    right = lax.rem(my + 1, N_DEV)

    barrier = pltpu.get_barrier_semaphore()
    for nbr in (left, right):
        pl.semaphore_signal(barrier, inc=1, device_id=(nbr,),
                            device_id_type=pl.DeviceIdType.MESH)
    pl.semaphore_wait(barrier, 2)

    xout_ref[my] = x_ref[...]
    rout_ref[my] = r_ref[...]

    for h in range(N_DEV - 1):
        origin = lax.rem(my + N_DEV - h, N_DEV)
        rdma_x = pltpu.make_async_remote_copy(
            src_ref=xout_ref.at[origin],
            dst_ref=xout_ref.at[origin],
            send_sem=x_send_sems.at[h],
            recv_sem=x_recv_sems.at[h],
            device_id=(right,),
            device_id_type=pl.DeviceIdType.MESH,
        )
        rdma_r = pltpu.make_async_remote_copy(
            src_ref=rout_ref.at[origin],
            dst_ref=rout_ref.at[origin],
            send_sem=r_send_sems.at[h],
            recv_sem=r_recv_sems.at[h],
            device_id=(right,),
            device_id_type=pl.DeviceIdType.MESH,
        )
        rdma_x.start()
        rdma_r.start()
        rdma_x.wait()
        rdma_r.wait()


def _all_gather(x_shard, router_t):
    return pl.pallas_call(
        _ag_body,
        out_shape=(
            jax.ShapeDtypeStruct((N_DEV, T_SHARD, D), jnp.float32),
            jax.ShapeDtypeStruct((N_DEV, E_LOCAL, D), jnp.float32),
        ),
        in_specs=[
            pl.BlockSpec(memory_space=pltpu.VMEM),
            pl.BlockSpec(memory_space=pltpu.VMEM),
        ],
        out_specs=(
            pl.BlockSpec(memory_space=pltpu.VMEM),
            pl.BlockSpec(memory_space=pltpu.VMEM),
        ),
        scratch_shapes=[
            pltpu.SemaphoreType.DMA((N_DEV - 1,)),
            pltpu.SemaphoreType.DMA((N_DEV - 1,)),
            pltpu.SemaphoreType.DMA((N_DEV - 1,)),
            pltpu.SemaphoreType.DMA((N_DEV - 1,)),
        ],
        compiler_params=pltpu.CompilerParams(collective_id=0),
    )(x_shard, router_t)


def _moe_body(x_ref, w1_ref, w2_ref, wt_ref, out_ref):
    e = pl.program_id(0)
    f = pl.program_id(1)

    @pl.when(jnp.logical_and(e == 0, f == 0))
    def _():
        out_ref[...] = jnp.zeros_like(out_ref)

    h = jnp.maximum(
        jnp.dot(x_ref[...], w1_ref[0], preferred_element_type=jnp.float32),
        0.0,
    )
    contrib = jnp.dot(h, w2_ref[0], preferred_element_type=jnp.float32)
    out_ref[...] += contrib * wt_ref[0]


def _expert_compute(x_full, W1, W2, wt):
    return pl.pallas_call(
        _moe_body,
        grid=(E_LOCAL, F // FBLK),
        in_specs=[
            pl.BlockSpec((T, D), lambda e, f: (0, 0)),
            pl.BlockSpec((1, D, FBLK), lambda e, f: (e, 0, f)),
            pl.BlockSpec((1, FBLK, D), lambda e, f: (e, f, 0)),
            pl.BlockSpec((1, T, 1), lambda e, f: (e, 0, 0)),
        ],
        out_specs=pl.BlockSpec((T, D), lambda e, f: (0, 0)),
        out_shape=jax.ShapeDtypeStruct((T, D), jnp.float32),
        compiler_params=pltpu.CompilerParams(
            dimension_semantics=("arbitrary", "arbitrary"),
        ),
    )(x_full, W1, W2, wt)


def _rs_body(ch_ref, out_ref, acc_ref, stage_ref, send_sems, recv_sems):
    my = lax.axis_index("i")
    left = lax.rem(my + N_DEV - 1, N_DEV)
    right = lax.rem(my + 1, N_DEV)

    barrier = pltpu.get_barrier_semaphore()
    for nbr in (left, right):
        pl.semaphore_signal(barrier, inc=1, device_id=(nbr,),
                            device_id_type=pl.DeviceIdType.MESH)
    pl.semaphore_wait(barrier, 2)

    for s in range(N_DEV - 1):
        if s == 0:
            src = ch_ref.at[0]
        else:
            stage_ref[s - 1] = acc_ref[s - 1] + ch_ref[s]
            src = stage_ref.at[s - 1]
        rdma = pltpu.make_async_remote_copy(
            src_ref=src,
            dst_ref=acc_ref.at[s],
            send_sem=send_sems.at[s],
            recv_sem=recv_sems.at[s],
            device_id=(right,),
            device_id_type=pl.DeviceIdType.MESH,
        )
        rdma.start()
        rdma.wait()

    out_ref[...] = acc_ref[N_DEV - 2] + ch_ref[N_DEV - 1]


def _reduce_scatter(chunks):
    return pl.pallas_call(
        _rs_body,
        out_shape=jax.ShapeDtypeStruct((T_SHARD, D), jnp.float32),
        in_specs=[pl.BlockSpec(memory_space=pltpu.VMEM)],
        out_specs=pl.BlockSpec(memory_space=pltpu.VMEM),
        scratch_shapes=[
            pltpu.VMEM((N_DEV - 1, T_SHARD, D), jnp.float32),
            pltpu.VMEM((N_DEV - 2, T_SHARD, D), jnp.float32),
            pltpu.SemaphoreType.DMA((N_DEV - 1,)),
            pltpu.SemaphoreType.DMA((N_DEV - 1,)),
        ],
        compiler_params=pltpu.CompilerParams(collective_id=1),
    )(chunks)


def kernel(x, router, W1, W2):
    my = lax.axis_index("i")

    xout, rout = _all_gather(x, router.T)
    x_full = xout.reshape(T, D)
    router_t_full = rout.reshape(E, D)

    gates = jnp.einsum("td,ed->te", x_full, router_t_full,
                       precision=lax.Precision.HIGHEST)
    top_vals, top_idx = lax.top_k(gates, 2)
    p2 = jnp.exp(top_vals[:, 1] - top_vals[:, 0])
    w1 = 1.0 / (1.0 + p2)
    w2 = p2 / (1.0 + p2)
    eids = jnp.arange(E, dtype=top_idx.dtype)
    w_full = ((top_idx[:, 0:1] == eids[None, :]) * w1[:, None]
              + (top_idx[:, 1:2] == eids[None, :]) * w2[:, None])
    w_full = w_full.astype(jnp.float32)
    w_local = lax.dynamic_slice(w_full, (0, my * E_LOCAL), (T, E_LOCAL))
    wt = w_local.T.reshape(E_LOCAL, T, 1)

    partial = _expert_compute(x_full, W1, W2, wt)

    rev = partial.reshape(N_DEV, T_SHARD, D)[::-1]
    doubled = jnp.concatenate([rev, rev], axis=0)
    chunks = lax.dynamic_slice(doubled, (N_DEV - my, 0, 0),
                               (N_DEV, T_SHARD, D))
    return _reduce_scatter(chunks)
